# Initial kernel scaffold; baseline (speedup 1.0000x reference)
import jax
import jax.numpy as jnp
from jax.experimental import pallas as pl


def kernel(input, flow, residual):
    raise NotImplementedError("write your pallas kernel here")


# speedup vs baseline: 7.0608x; 7.0608x over previous
import functools

import jax
import jax.numpy as jnp
from jax import lax
from jax.experimental import pallas as pl
from jax.experimental.pallas import tpu as pltpu
from jax.experimental.pallas import tpu_sc as plsc

_NC = 2
_NS = 16
_NW = _NC * _NS
_K = 128
_L = 16


def _interp_sc(xt, idx4, w4, C):
    N, CP = xt.shape
    per_w = N // _NW
    chunks = per_w // _K
    mesh = plsc.VectorSubcoreMesh(core_axis_name="c", subcore_axis_name="s")

    @functools.partial(
        pl.kernel,
        out_type=jax.ShapeDtypeStruct((N, C), jnp.float32),
        mesh=mesh,
        scratch_types=[
            pltpu.VMEM((4, _K), jnp.int32),
            pltpu.VMEM((4, _K), jnp.float32),
            pltpu.VMEM((_K, CP), jnp.float32),
            pltpu.VMEM((_K, CP), jnp.float32),
            pltpu.VMEM((_K, CP), jnp.float32),
            pltpu.VMEM((_K, CP), jnp.float32),
            pltpu.VMEM((_K, C), jnp.float32),
            pltpu.SemaphoreType.DMA,
        ],
    )
    def k(xt_hbm, idx_hbm, w_hbm, out_hbm, idx_v, w_v,
          tap0_v, tap1_v, tap2_v, tap3_v, out_v, sem):
        taps = (tap0_v, tap1_v, tap2_v, tap3_v)
        wid = lax.axis_index("s") * _NC + lax.axis_index("c")
        base0 = wid * per_w

        def chunk_body(g, carry):
            base = base0 + g * _K
            for t in range(4):
                pltpu.sync_copy(idx_hbm.at[t, pl.ds(base, _K)], idx_v.at[t])
                pltpu.sync_copy(w_hbm.at[t, pl.ds(base, _K)], w_v.at[t])
            copies = [
                pltpu.async_copy(xt_hbm.at[idx_v.at[t]], taps[t], sem)
                for t in range(4)
            ]
            for c in copies:
                c.wait()

            def grp_body(g2, carry2):
                base_p = g2 * _L
                wv = [w_v[t, pl.ds(base_p, _L)] for t in range(4)]
                for i in range(_L):
                    p = base_p + i
                    w0, w1, w2, w3 = wv[0][i], wv[1][i], wv[2][i], wv[3][i]
                    for j in range(C // _L):
                        s = pl.ds(j * _L, _L)
                        acc = w0 * tap0_v[p, s] + w1 * tap1_v[p, s]
                        acc = acc + w2 * tap2_v[p, s] + w3 * tap3_v[p, s]
                        out_v[p, s] = acc
                return carry2

            lax.fori_loop(0, _K // _L, grp_body, 0)
            pltpu.sync_copy(out_v, out_hbm.at[pl.ds(base, _K)])
            return carry

        lax.fori_loop(0, chunks, chunk_body, 0, unroll=False)

    return k(xt, idx4, w4)


def kernel(input, flow, residual):
    x = input
    B, C, H, W = x.shape
    N = B * H * W

    gy = jnp.linspace(-1.0 + 1.0 / H, 1.0 - 1.0 / H, H, dtype=x.dtype)
    gx = jnp.linspace(-1.0 + 1.0 / W, 1.0 - 1.0 / W, W, dtype=x.dtype)
    grid_x = gx[None, None, :] + flow[:, 0]
    grid_y = gy[None, :, None] + flow[:, 1]
    grid_x = jnp.remainder(grid_x + 1.0, 2.0) - 1.0
    real_x = (grid_x + 1.0) * (W * 0.5) - 0.5
    real_y = (grid_y + 1.0) * (H * 0.5) - 0.5
    x0f = jnp.floor(real_x)
    y0f = jnp.floor(real_y)
    dx = real_x - x0f
    dy = real_y - y0f
    ix0 = x0f.astype(jnp.int32)
    iy0 = y0f.astype(jnp.int32)
    boff = (jnp.arange(B, dtype=jnp.int32) * (H * W))[:, None, None]

    def idx_w(iy, ix, w):
        valid = (ix >= 0) & (ix < W) & (iy >= 0) & (iy < H)
        iyc = jnp.clip(iy, 0, H - 1)
        ixc = jnp.clip(ix, 0, W - 1)
        idx = iyc * W + ixc + boff
        return idx.reshape(N), (w * valid).reshape(N)

    i_tl, w_tl = idx_w(iy0, ix0, (1.0 - dx) * (1.0 - dy))
    i_tr, w_tr = idx_w(iy0, ix0 + 1, dx * (1.0 - dy))
    i_bl, w_bl = idx_w(iy0 + 1, ix0, (1.0 - dx) * dy)
    i_br, w_br = idx_w(iy0 + 1, ix0 + 1, dx * dy)
    idx4 = jnp.stack([i_tl, i_tr, i_bl, i_br])
    w4 = jnp.stack([w_tl, w_tr, w_bl, w_br])

    xt = jnp.transpose(x, (0, 2, 3, 1)).reshape(N, C)
    xt = jnp.pad(xt, ((0, 0), (0, 128 - C)))
    out_t = _interp_sc(xt, idx4, w4, C)
    return out_t.reshape(B, H, W, C).transpose(0, 3, 1, 2) + residual


# speedup vs baseline: 7.6798x; 1.0877x over previous
import functools

import jax
import jax.numpy as jnp
from jax import lax
from jax.experimental import pallas as pl
from jax.experimental.pallas import tpu as pltpu
from jax.experimental.pallas import tpu_sc as plsc

_NC = 2
_NS = 16
_NW = _NC * _NS
_K = 64
_L = 16
_CP = 128


def _interp_sc(xt, meta, wts, N, C):
    per_w = N // _NW
    chunks = per_w // _K
    mesh = plsc.VectorSubcoreMesh(core_axis_name="c", subcore_axis_name="s")

    taps_t = pltpu.VMEM((_K, _CP), jnp.float32)

    @functools.partial(
        pl.kernel,
        out_type=jax.ShapeDtypeStruct((N, C), jnp.float32),
        mesh=mesh,
        scratch_types=[
            pltpu.VMEM((4, _K), jnp.int32),
            pltpu.VMEM((4, _K), jnp.int32),
            pltpu.VMEM((4, _K), jnp.float32),
            taps_t, taps_t, taps_t, taps_t,
            taps_t, taps_t, taps_t, taps_t,
            pltpu.VMEM((_K, C), jnp.float32),
            pltpu.SemaphoreType.DMA,
            pltpu.SemaphoreType.DMA,
        ],
    )
    def k(xt_hbm, meta_hbm, w_hbm, out_hbm, m0, m1, w_v,
          ta0, ta1, ta2, ta3, tb0, tb1, tb2, tb3, out_v, sem_a, sem_b):
        wid = lax.axis_index("s") * _NC + lax.axis_index("c")
        base0 = wid * per_w
        metas = (m0, m1)
        taps = ((ta0, ta1, ta2, ta3), (tb0, tb1, tb2, tb3))
        sems = (sem_a, sem_b)

        def wait_gathers(p):
            for t in range(4):
                pltpu.make_async_copy(
                    xt_hbm.at[metas[p].at[t]], taps[p][t], sems[p]).wait()

        def issue_gathers(p):
            for t in range(4):
                pltpu.async_copy(
                    xt_hbm.at[metas[p].at[t]], taps[p][t], sems[p])

        def do_chunk(g, p):
            q = 1 - p
            wait_gathers(p)
            pltpu.sync_copy(meta_hbm.at[wid, g + 1], metas[q])
            issue_gathers(q)
            pltpu.sync_copy(w_hbm.at[wid, g], w_v)
            tp = taps[p]

            def grp_body(g2, carry):
                bp = g2 * _L
                wv = [w_v[t, pl.ds(bp, _L)] for t in range(4)]
                for i in range(_L):
                    pix = bp + i
                    ws = [wv[t][i] for t in range(4)]
                    for j in range(C // _L):
                        s = pl.ds(j * _L, _L)
                        acc = ws[0] * tp[0][pix, s] + ws[1] * tp[1][pix, s]
                        acc = acc + ws[2] * tp[2][pix, s] + ws[3] * tp[3][pix, s]
                        out_v[pix, s] = acc
                return carry

            lax.fori_loop(0, _K // _L, grp_body, 0)
            pltpu.sync_copy(out_v, out_hbm.at[pl.ds(base0 + g * _K, _K)])

        pltpu.sync_copy(meta_hbm.at[wid, 0], m0)
        issue_gathers(0)

        def pair_body(i, carry):
            do_chunk(2 * i, 0)
            do_chunk(2 * i + 1, 1)
            return carry

        lax.fori_loop(0, chunks // 2, pair_body, 0)
        wait_gathers(0)

    return k(xt, meta, wts)


def kernel(input, flow, residual):
    x = input
    B, C, H, W = x.shape
    N = B * H * W
    per_w = N // _NW
    chunks = per_w // _K

    gy = jnp.linspace(-1.0 + 1.0 / H, 1.0 - 1.0 / H, H, dtype=x.dtype)
    gx = jnp.linspace(-1.0 + 1.0 / W, 1.0 - 1.0 / W, W, dtype=x.dtype)
    grid_x = gx[None, None, :] + flow[:, 0]
    grid_y = gy[None, :, None] + flow[:, 1]
    grid_x = jnp.remainder(grid_x + 1.0, 2.0) - 1.0
    real_x = (grid_x + 1.0) * (W * 0.5) - 0.5
    real_y = (grid_y + 1.0) * (H * 0.5) - 0.5
    x0f = jnp.floor(real_x)
    y0f = jnp.floor(real_y)
    dx = real_x - x0f
    dy = real_y - y0f
    ix0 = x0f.astype(jnp.int32)
    iy0 = y0f.astype(jnp.int32)
    boff = (jnp.arange(B, dtype=jnp.int32) * (H * W))[:, None, None]

    def idx_w(iy, ix, w):
        valid = (ix >= 0) & (ix < W) & (iy >= 0) & (iy < H)
        iyc = jnp.clip(iy, 0, H - 1)
        ixc = jnp.clip(ix, 0, W - 1)
        idx = iyc * W + ixc + boff
        return idx.reshape(N), (w * valid).reshape(N)

    i_tl, w_tl = idx_w(iy0, ix0, (1.0 - dx) * (1.0 - dy))
    i_tr, w_tr = idx_w(iy0, ix0 + 1, dx * (1.0 - dy))
    i_bl, w_bl = idx_w(iy0 + 1, ix0, (1.0 - dx) * dy)
    i_br, w_br = idx_w(iy0 + 1, ix0 + 1, dx * dy)
    idx4 = jnp.stack([i_tl, i_tr, i_bl, i_br])
    w4 = jnp.stack([w_tl, w_tr, w_bl, w_br])
    meta = idx4.reshape(4, _NW, chunks, _K).transpose(1, 2, 0, 3)
    meta = jnp.pad(meta, ((0, 0), (0, 1), (0, 0), (0, 0)))
    wts = w4.reshape(4, _NW, chunks, _K).transpose(1, 2, 0, 3)

    xt = jnp.transpose(x, (0, 2, 3, 1)).reshape(N, C)
    xt = jnp.pad(xt, ((0, 0), (0, _CP - C)))
    out_t = _interp_sc(xt, meta, wts, N, C)
    return out_t.reshape(B, H, W, C).transpose(0, 3, 1, 2) + residual


# speedup vs baseline: 8.7787x; 1.1431x over previous
import functools

import jax
import jax.numpy as jnp
from jax import lax
from jax.experimental import pallas as pl
from jax.experimental.pallas import tpu as pltpu
from jax.experimental.pallas import tpu_sc as plsc

_NC = 2
_NS = 16
_NW = _NC * _NS
_K = 64
_L = 16
_CP = 128


def _interp_sc(xt, meta, wts, N, C):
    per_w = N // _NW
    chunks = per_w // _K
    mesh = plsc.VectorSubcoreMesh(core_axis_name="c", subcore_axis_name="s")

    taps_t = pltpu.VMEM((_K, _CP), jnp.float32)

    @functools.partial(
        pl.kernel,
        out_type=jax.ShapeDtypeStruct((N, C), jnp.float32),
        mesh=mesh,
        scratch_types=[
            pltpu.VMEM((4, _K), jnp.int32),
            pltpu.VMEM((4, _K), jnp.int32),
            pltpu.VMEM((4, _K), jnp.float32),
            pltpu.VMEM((4, _K), jnp.float32),
            taps_t, taps_t, taps_t, taps_t,
            taps_t, taps_t, taps_t, taps_t,
            pltpu.VMEM((_K, C), jnp.float32),
            pltpu.VMEM((_K, C), jnp.float32),
            pltpu.SemaphoreType.DMA,
            pltpu.SemaphoreType.DMA,
            pltpu.SemaphoreType.DMA,
            pltpu.SemaphoreType.DMA,
            pltpu.SemaphoreType.DMA,
            pltpu.SemaphoreType.DMA,
            pltpu.SemaphoreType.DMA,
            pltpu.SemaphoreType.DMA,
        ],
    )
    def k(xt_hbm, meta_hbm, w_hbm, out_hbm, m0, m1, w0, w1,
          ta0, ta1, ta2, ta3, tb0, tb1, tb2, tb3, ov0, ov1,
          sem_ga, sem_gb, sem_ma, sem_mb, sem_wa, sem_wb, sem_oa, sem_ob):
        wid = lax.axis_index("s") * _NC + lax.axis_index("c")
        base0 = wid * per_w
        metas = (m0, m1)
        wvs = (w0, w1)
        taps = ((ta0, ta1, ta2, ta3), (tb0, tb1, tb2, tb3))
        outs = (ov0, ov1)
        gsems = (sem_ga, sem_gb)
        msems = (sem_ma, sem_mb)
        wsems = (sem_wa, sem_wb)
        osems = (sem_oa, sem_ob)

        def wait_gathers(p):
            for t in range(4):
                pltpu.make_async_copy(
                    xt_hbm.at[metas[p].at[t]], taps[p][t], gsems[p]).wait()

        def issue_gathers(p):
            for t in range(4):
                pltpu.async_copy(
                    xt_hbm.at[metas[p].at[t]], taps[p][t], gsems[p])

        def issue_meta(g, p):
            pltpu.async_copy(meta_hbm.at[wid, g], metas[p], msems[p])

        def wait_meta(p):
            pltpu.make_async_copy(
                meta_hbm.at[wid, 0], metas[p], msems[p]).wait()

        def issue_w(g, p):
            pltpu.async_copy(w_hbm.at[wid, g], wvs[p], wsems[p])

        def wait_w(p):
            pltpu.make_async_copy(
                w_hbm.at[wid, 0], wvs[p], wsems[p]).wait()

        def wait_out(p):
            pltpu.make_async_copy(
                outs[p], out_hbm.at[pl.ds(0, _K)], osems[p]).wait()

        def do_chunk(g, i, p, guard_out):
            q = 1 - p
            wait_gathers(p)
            wait_meta(q)
            issue_gathers(q)
            issue_meta(g + 2, p)
            wait_w(p)
            if guard_out is None:
                wait_out(p)
            else:
                @pl.when(guard_out)
                def _():
                    wait_out(p)
            tp = taps[p]
            w_v = wvs[p]
            out_v = outs[p]

            def grp_body(g2, carry):
                bp = g2 * _L
                wv = [w_v[t, pl.ds(bp, _L)] for t in range(4)]
                for ii in range(_L):
                    pix = bp + ii
                    ws = [wv[t][ii] for t in range(4)]
                    for j in range(C // _L):
                        s = pl.ds(j * _L, _L)
                        acc = ws[0] * tp[0][pix, s] + ws[1] * tp[1][pix, s]
                        acc = acc + ws[2] * tp[2][pix, s] + ws[3] * tp[3][pix, s]
                        out_v[pix, s] = acc
                return carry

            lax.fori_loop(0, _K // _L, grp_body, 0)
            issue_w(g + 2, p)
            pltpu.async_copy(
                out_v, out_hbm.at[pl.ds(base0 + g * _K, _K)], osems[p])

        issue_meta(0, 0)
        issue_meta(1, 1)
        issue_w(0, 0)
        issue_w(1, 1)
        wait_meta(0)
        issue_gathers(0)

        def pair_body(i, carry):
            do_chunk(2 * i, i, 0, i > 0)
            do_chunk(2 * i + 1, i, 1, i > 0)
            return carry

        lax.fori_loop(0, chunks // 2, pair_body, 0)
        wait_gathers(0)
        wait_meta(1)
        wait_w(0)
        wait_w(1)
        wait_out(0)
        wait_out(1)

    return k(xt, meta, wts)


def kernel(input, flow, residual):
    x = input
    B, C, H, W = x.shape
    N = B * H * W
    per_w = N // _NW
    chunks = per_w // _K

    gy = jnp.linspace(-1.0 + 1.0 / H, 1.0 - 1.0 / H, H, dtype=x.dtype)
    gx = jnp.linspace(-1.0 + 1.0 / W, 1.0 - 1.0 / W, W, dtype=x.dtype)
    grid_x = gx[None, None, :] + flow[:, 0]
    grid_y = gy[None, :, None] + flow[:, 1]
    grid_x = jnp.remainder(grid_x + 1.0, 2.0) - 1.0
    real_x = (grid_x + 1.0) * (W * 0.5) - 0.5
    real_y = (grid_y + 1.0) * (H * 0.5) - 0.5
    x0f = jnp.floor(real_x)
    y0f = jnp.floor(real_y)
    dx = real_x - x0f
    dy = real_y - y0f
    ix0 = x0f.astype(jnp.int32)
    iy0 = y0f.astype(jnp.int32)
    boff = (jnp.arange(B, dtype=jnp.int32) * (H * W))[:, None, None]

    def idx_w(iy, ix, w):
        valid = (ix >= 0) & (ix < W) & (iy >= 0) & (iy < H)
        iyc = jnp.clip(iy, 0, H - 1)
        ixc = jnp.clip(ix, 0, W - 1)
        idx = iyc * W + ixc + boff
        return idx.reshape(N), (w * valid).reshape(N)

    i_tl, w_tl = idx_w(iy0, ix0, (1.0 - dx) * (1.0 - dy))
    i_tr, w_tr = idx_w(iy0, ix0 + 1, dx * (1.0 - dy))
    i_bl, w_bl = idx_w(iy0 + 1, ix0, (1.0 - dx) * dy)
    i_br, w_br = idx_w(iy0 + 1, ix0 + 1, dx * dy)
    idx4 = jnp.stack([i_tl, i_tr, i_bl, i_br])
    w4 = jnp.stack([w_tl, w_tr, w_bl, w_br])
    meta = idx4.reshape(4, _NW, chunks, _K).transpose(1, 2, 0, 3)
    meta = jnp.pad(meta, ((0, 0), (0, 2), (0, 0), (0, 0)))
    wts = w4.reshape(4, _NW, chunks, _K).transpose(1, 2, 0, 3)
    wts = jnp.pad(wts, ((0, 0), (0, 2), (0, 0), (0, 0)))

    xt = jnp.transpose(x, (0, 2, 3, 1)).reshape(N, C)
    xt = jnp.pad(xt, ((0, 0), (0, _CP - C)))
    out_t = _interp_sc(xt, meta, wts, N, C)
    return out_t.reshape(B, H, W, C).transpose(0, 3, 1, 2) + residual


# speedup vs baseline: 8.7892x; 1.0012x over previous
import functools

import jax
import jax.numpy as jnp
from jax import lax
from jax.experimental import pallas as pl
from jax.experimental.pallas import tpu as pltpu
from jax.experimental.pallas import tpu_sc as plsc

_NC = 2
_NS = 16
_NW = _NC * _NS
_K = 64
_L = 16
_CP = 128


def _interp_sc(xt, meta, wts, N, C):
    per_w = N // _NW
    chunks = per_w // _K
    mesh = plsc.VectorSubcoreMesh(core_axis_name="c", subcore_axis_name="s")

    taps_t = pltpu.VMEM((_K, _CP), jnp.float32)

    @functools.partial(
        pl.kernel,
        out_type=jax.ShapeDtypeStruct((N, C), jnp.float32),
        mesh=mesh,
        compiler_params=pltpu.CompilerParams(use_tc_tiling_on_sc=True),
        scratch_types=[
            pltpu.VMEM((4, _K), jnp.int32),
            pltpu.VMEM((4, _K), jnp.int32),
            pltpu.VMEM((4, _K), jnp.float32),
            pltpu.VMEM((4, _K), jnp.float32),
            taps_t, taps_t, taps_t, taps_t,
            taps_t, taps_t, taps_t, taps_t,
            pltpu.VMEM((_K, C), jnp.float32),
            pltpu.VMEM((_K, C), jnp.float32),
            pltpu.SemaphoreType.DMA,
            pltpu.SemaphoreType.DMA,
            pltpu.SemaphoreType.DMA,
            pltpu.SemaphoreType.DMA,
            pltpu.SemaphoreType.DMA,
            pltpu.SemaphoreType.DMA,
            pltpu.SemaphoreType.DMA,
            pltpu.SemaphoreType.DMA,
        ],
    )
    def k(xt_hbm, meta_hbm, w_hbm, out_hbm, m0, m1, w0, w1,
          ta0, ta1, ta2, ta3, tb0, tb1, tb2, tb3, ov0, ov1,
          sem_ga, sem_gb, sem_ma, sem_mb, sem_wa, sem_wb, sem_oa, sem_ob):
        wid = lax.axis_index("s") * _NC + lax.axis_index("c")
        base0 = wid * per_w
        metas = (m0, m1)
        wvs = (w0, w1)
        taps = ((ta0, ta1, ta2, ta3), (tb0, tb1, tb2, tb3))
        outs = (ov0, ov1)
        gsems = (sem_ga, sem_gb)
        msems = (sem_ma, sem_mb)
        wsems = (sem_wa, sem_wb)
        osems = (sem_oa, sem_ob)

        def wait_gathers(p):
            for t in range(4):
                pltpu.make_async_copy(
                    xt_hbm.at[metas[p].at[t]], taps[p][t], gsems[p]).wait()

        def issue_gathers(p):
            for t in range(4):
                pltpu.async_copy(
                    xt_hbm.at[metas[p].at[t]], taps[p][t], gsems[p])

        def issue_meta(g, p):
            pltpu.async_copy(meta_hbm.at[wid, g], metas[p], msems[p])

        def wait_meta(p):
            pltpu.make_async_copy(
                meta_hbm.at[wid, 0], metas[p], msems[p]).wait()

        def issue_w(g, p):
            pltpu.async_copy(w_hbm.at[wid, g], wvs[p], wsems[p])

        def wait_w(p):
            pltpu.make_async_copy(
                w_hbm.at[wid, 0], wvs[p], wsems[p]).wait()

        def wait_out(p):
            pltpu.make_async_copy(
                outs[p], out_hbm.at[pl.ds(0, _K)], osems[p]).wait()

        def do_chunk(g, i, p, guard_out):
            q = 1 - p
            wait_gathers(p)
            wait_meta(q)
            issue_gathers(q)
            issue_meta(g + 2, p)
            wait_w(p)
            if guard_out is None:
                wait_out(p)
            else:
                @pl.when(guard_out)
                def _():
                    wait_out(p)
            tp = taps[p]
            w_v = wvs[p]
            out_v = outs[p]

            def grp_body(g2, carry):
                bp = g2 * _L
                wv = [w_v[t, pl.ds(bp, _L)] for t in range(4)]
                for ii in range(_L):
                    pix = bp + ii
                    ws = [wv[t][ii] for t in range(4)]
                    for j in range(C // _L):
                        s = pl.ds(j * _L, _L)
                        acc = ws[0] * tp[0][pix, s] + ws[1] * tp[1][pix, s]
                        acc = acc + ws[2] * tp[2][pix, s] + ws[3] * tp[3][pix, s]
                        out_v[pix, s] = acc
                return carry

            lax.fori_loop(0, _K // _L, grp_body, 0)
            issue_w(g + 2, p)
            pltpu.async_copy(
                out_v, out_hbm.at[pl.ds(base0 + g * _K, _K)], osems[p])

        issue_meta(0, 0)
        issue_meta(1, 1)
        issue_w(0, 0)
        issue_w(1, 1)
        wait_meta(0)
        issue_gathers(0)

        def pair_body(i, carry):
            do_chunk(2 * i, i, 0, i > 0)
            do_chunk(2 * i + 1, i, 1, i > 0)
            return carry

        lax.fori_loop(0, chunks // 2, pair_body, 0)
        wait_gathers(0)
        wait_meta(1)
        wait_w(0)
        wait_w(1)
        wait_out(0)
        wait_out(1)

    return k(xt, meta, wts)


def kernel(input, flow, residual):
    x = input
    B, C, H, W = x.shape
    N = B * H * W
    per_w = N // _NW
    chunks = per_w // _K

    gy = jnp.linspace(-1.0 + 1.0 / H, 1.0 - 1.0 / H, H, dtype=x.dtype)
    gx = jnp.linspace(-1.0 + 1.0 / W, 1.0 - 1.0 / W, W, dtype=x.dtype)
    grid_x = gx[None, None, :] + flow[:, 0]
    grid_y = gy[None, :, None] + flow[:, 1]
    grid_x = jnp.remainder(grid_x + 1.0, 2.0) - 1.0
    real_x = (grid_x + 1.0) * (W * 0.5) - 0.5
    real_y = (grid_y + 1.0) * (H * 0.5) - 0.5
    x0f = jnp.floor(real_x)
    y0f = jnp.floor(real_y)
    dx = real_x - x0f
    dy = real_y - y0f
    ix0 = x0f.astype(jnp.int32)
    iy0 = y0f.astype(jnp.int32)
    boff = (jnp.arange(B, dtype=jnp.int32) * (H * W))[:, None, None]

    def idx_w(iy, ix, w):
        valid = (ix >= 0) & (ix < W) & (iy >= 0) & (iy < H)
        iyc = jnp.clip(iy, 0, H - 1)
        ixc = jnp.clip(ix, 0, W - 1)
        idx = iyc * W + ixc + boff
        return idx.reshape(N), (w * valid).reshape(N)

    i_tl, w_tl = idx_w(iy0, ix0, (1.0 - dx) * (1.0 - dy))
    i_tr, w_tr = idx_w(iy0, ix0 + 1, dx * (1.0 - dy))
    i_bl, w_bl = idx_w(iy0 + 1, ix0, (1.0 - dx) * dy)
    i_br, w_br = idx_w(iy0 + 1, ix0 + 1, dx * dy)
    idx4 = jnp.stack([i_tl, i_tr, i_bl, i_br])
    w4 = jnp.stack([w_tl, w_tr, w_bl, w_br])
    meta = idx4.reshape(4, _NW, chunks, _K).transpose(1, 2, 0, 3)
    meta = jnp.pad(meta, ((0, 0), (0, 2), (0, 0), (0, 0)))
    wts = w4.reshape(4, _NW, chunks, _K).transpose(1, 2, 0, 3)
    wts = jnp.pad(wts, ((0, 0), (0, 2), (0, 0), (0, 0)))

    xt = jnp.transpose(x, (0, 2, 3, 1)).reshape(N, C)
    xt = jnp.pad(xt, ((0, 0), (0, _CP - C)))
    out_t = _interp_sc(xt, meta, wts, N, C)
    return out_t.reshape(B, H, W, C).transpose(0, 3, 1, 2) + residual
